# Initial kernel scaffold; baseline (speedup 1.0000x reference)
import jax
import jax.numpy as jnp
from jax.experimental import pallas as pl


def kernel(combined_data, ico_positions, input_means, input_stds, We1, be1, ge1, oe1, We2, be2, ge2, oe2, Wn1, bn1, gn1, on1, Wn2, bn2, gn2, on2):
    raise NotImplementedError("write your pallas kernel here")


# speedup vs baseline: 185.8293x; 185.8293x over previous
import functools

import jax
import jax.numpy as jnp
from jax import lax
from jax.experimental import pallas as pl
from jax.experimental.pallas import tpu as pltpu
from jax.experimental.pallas import tpu_sc as plsc

_R_EARTH = 6371.0
_MASK_KM = 82.5
_N = 5978
_K = 32
_H = 361
_W = 720
_NPAD = 6016
_NB = 128
_GRID = _NPAD // _NB
_NROW = 9
_NCOL = 64
_WIN = 640
_NE = _NPAD * _K
_EB = _NB * _K


def _knn_body(slat_ref, cc_ref, slon_ref, fidx_ref, oi_ref, oa_ref):
    a = slat_ref[...] + cc_ref[...] * slon_ref[...]
    fidx = fidx_ref[...]
    kiota = lax.broadcasted_iota(jnp.int32, (_NB, _K), 1)
    outi = jnp.zeros((_NB, _K), jnp.float32)
    outa = jnp.zeros((_NB, _K), jnp.float32)
    for k in range(_K):
        m = jnp.min(a, axis=1, keepdims=True)
        cand = jnp.where(a == m, fidx, 1e9)
        sel = jnp.min(cand, axis=1, keepdims=True)
        outi = jnp.where(kiota == k, sel, outi)
        outa = jnp.where(kiota == k, m, outa)
        a = jnp.where(fidx == sel, 1e30, a)
    oi_ref[...] = outi
    oa_ref[...] = outa


def _edge_body(x_ref, w1_ref, b1_ref, g1_ref, o1_ref,
               w2_ref, b2_ref, g2_ref, o2_ref, out_ref):
    x = x_ref[...]
    d = x[2:3, :]
    mask = (d <= _MASK_KM).astype(jnp.float32)
    c0 = x[0:1, :] - x[3:4, :]
    c1 = x[1:2, :] - x[4:5, :]
    c3 = jnp.cos(x[0:1, :] - x[5:6, :])
    c4 = jnp.sin(x[1:2, :] - x[6:7, :])
    zero = jnp.zeros_like(c0)
    c = jnp.concatenate([c0, c1, d, c3, c4, zero, zero, zero], axis=0)
    h = jnp.dot(w1_ref[...], c, preferred_element_type=jnp.float32)
    h = jnp.maximum(h + b1_ref[...], 0.0)
    m1 = jnp.mean(h, axis=0, keepdims=True)
    v1 = jnp.mean((h - m1) ** 2, axis=0, keepdims=True)
    h = (h - m1) / jnp.sqrt(v1 + 1e-5) * g1_ref[...] + o1_ref[...]
    h = jnp.dot(w2_ref[...], h, preferred_element_type=jnp.float32)
    h = jnp.maximum(h + b2_ref[...], 0.0)
    m2 = jnp.mean(h, axis=0, keepdims=True)
    v2 = jnp.mean((h - m2) ** 2, axis=0, keepdims=True)
    h = (h - m2) / jnp.sqrt(v2 + 1e-5) * g2_ref[...] + o2_ref[...]
    out_ref[...] = h * mask


def _node_body(d_ref, g_ref, mu_ref, sg_ref, w1_ref, b1_ref, g1_ref, o1_ref,
               w2_ref, b2_ref, g2_ref, o2_ref, out_ref):
    d = d_ref[...]
    mask = (d <= _MASK_KM).astype(jnp.float32)
    w = jnp.exp(-d / _MASK_KM) * mask
    sw = jnp.sum(w, axis=1, keepdims=True)
    w = w / (sw + 1e-7)
    swn = jnp.sum(w, axis=1, keepdims=True)
    agg = jnp.sum(g_ref[...] * w[:, :, None], axis=1)
    x = (agg - mu_ref[...] * swn) / (sg_ref[...] + 1e-7)
    h = jnp.dot(x, w1_ref[...], preferred_element_type=jnp.float32)
    h = jnp.maximum(h + b1_ref[...], 0.0)
    m1 = jnp.mean(h, axis=1, keepdims=True)
    v1 = jnp.mean((h - m1) ** 2, axis=1, keepdims=True)
    h = (h - m1) / jnp.sqrt(v1 + 1e-5) * g1_ref[...] + o1_ref[...]
    h = jnp.dot(h, w2_ref[...], preferred_element_type=jnp.float32)
    h = jnp.maximum(h + b2_ref[...], 0.0)
    m2 = jnp.mean(h, axis=1, keepdims=True)
    v2 = jnp.mean((h - m2) ** 2, axis=1, keepdims=True)
    h = (h - m2) / jnp.sqrt(v2 + 1e-5) * g2_ref[...] + o2_ref[...]
    out_ref[...] = h


def _sc_gather(table, idx):
    n_idx = idx.shape[0]
    n_workers = 32
    b_per_w = n_idx // n_workers
    chunk = 376
    n_chunks = b_per_w // chunk
    mesh = plsc.VectorSubcoreMesh(core_axis_name="c", subcore_axis_name="s")

    @functools.partial(
        pl.kernel, mesh=mesh,
        out_type=jax.ShapeDtypeStruct((n_idx, 128), jnp.float32),
        scratch_types=[
            pltpu.VMEM((chunk,), jnp.int32),
            pltpu.VMEM((chunk, 128), jnp.float32),
            pltpu.SemaphoreType.DMA,
        ],
    )
    def k(table_hbm, idx_hbm, out_hbm, idx_v, rows_v, sem):
        wid = lax.axis_index("s") * 2 + lax.axis_index("c")
        base = wid * b_per_w

        @pl.loop(0, n_chunks)
        def _(i):
            off = base + i * chunk
            pltpu.sync_copy(idx_hbm.at[pl.ds(off, chunk)], idx_v)
            pltpu.async_copy(table_hbm.at[idx_v], rows_v, sem).wait()
            pltpu.sync_copy(rows_v, out_hbm.at[pl.ds(off, chunk)])

    return k(table, idx)


def kernel(combined_data, ico_positions, input_means, input_stds,
           We1, be1, ge1, oe1, We2, be2, ge2, oe2,
           Wn1, bn1, gn1, on1, Wn2, bn2, gn2, on2):
    f32 = jnp.float32
    lat_grid = jnp.radians(jnp.linspace(-90.0, 90.0, _H, dtype=f32))
    lon_grid = jnp.radians(jnp.linspace(0.0, 360.0, _W, dtype=f32))
    ico_lat = jnp.radians(ico_positions[:, 0])
    ico_lon = jnp.radians(ico_positions[:, 1])
    slat2 = jnp.sin((lat_grid[None, :] - ico_lat[:, None]) / 2) ** 2
    cc = jnp.cos(ico_lat)[:, None] * jnp.cos(lat_grid)[None, :]
    slon2 = jnp.sin((lon_grid[None, :] - ico_lon[:, None]) / 2) ** 2

    crow = jnp.argmin(slat2, axis=1).astype(jnp.int32)
    start = jnp.clip(crow - 4, 0, _H - _NROW)
    ccol = jnp.argmin(slon2, axis=1).astype(jnp.int32)
    colmap = jnp.mod(ccol[:, None] - 31 + jnp.arange(63, dtype=jnp.int32)[None, :], 719)
    colmap = jnp.concatenate(
        [colmap, jnp.full((_N, 1), _W - 1, jnp.int32)], axis=1)
    rows = start[:, None] + jnp.arange(_NROW, dtype=jnp.int32)[None, :]
    slat2w = jnp.take_along_axis(slat2, rows, axis=1)
    ccw = jnp.take_along_axis(cc, rows, axis=1)
    slon2w = jnp.take_along_axis(slon2, colmap, axis=1)

    big = jnp.float32(1e9)
    slat_t = jnp.concatenate(
        [jnp.repeat(slat2w, _NCOL, axis=1), jnp.full((_N, _NCOL), big, f32)], axis=1)
    cc_t = jnp.concatenate(
        [jnp.repeat(ccw, _NCOL, axis=1), jnp.zeros((_N, _NCOL), f32)], axis=1)
    slon_t = jnp.tile(slon2w, (1, _NROW + 1))
    fidx_t = jnp.concatenate(
        [jnp.repeat((rows * _W).astype(f32), _NCOL, axis=1)
         + jnp.tile(colmap.astype(f32), (1, _NROW)),
         300000.0 + jnp.arange(_NCOL, dtype=f32)[None, :]
         + jnp.zeros((_N, 1), f32)], axis=1)

    def padn(x):
        return jnp.pad(x, ((0, _NPAD - _N),) + ((0, 0),) * (x.ndim - 1), mode="edge")

    slat_t, cc_t, slon_t, fidx_t = map(padn, (slat_t, cc_t, slon_t, fidx_t))

    bspec_in = pl.BlockSpec((_NB, _WIN), lambda i: (i, 0))
    bspec_out = pl.BlockSpec((_NB, _K), lambda i: (i, 0))
    sel_f, sel_a = pl.pallas_call(
        _knn_body,
        grid=(_GRID,),
        in_specs=[bspec_in] * 4,
        out_specs=[bspec_out, bspec_out],
        out_shape=[jax.ShapeDtypeStruct((_NPAD, _K), f32)] * 2,
    )(slat_t, cc_t, slon_t, fidx_t)

    fi = sel_f.astype(jnp.int32)
    row = fi // _W
    col = fi - row * _W
    llat = jnp.take(lat_grid, row)
    llon = jnp.take(lon_grid, col)
    central = 2.0 * jnp.arcsin(jnp.sqrt(jnp.clip(sel_a, 0.0, 1.0)))
    d = central * _R_EARTH

    table = jnp.pad(combined_data.reshape(78, -1), ((0, 50), (0, 0))).T
    gathered = _sc_gather(table, fi.reshape(-1))

    deg_la = jnp.degrees(ico_lat)
    deg_lo = jnp.degrees(ico_lon)

    def pedge(node_arr):
        return jnp.repeat(jnp.pad(node_arr, (0, _NPAD - _N), mode="edge"), _K)

    X = jnp.stack([llat.reshape(-1), llon.reshape(-1), d.reshape(-1),
                   pedge(deg_la), pedge(deg_lo), pedge(ico_lat), pedge(ico_lon),
                   jnp.zeros((_NE,), f32)], axis=0)
    w1e = jnp.concatenate([We1.T, jnp.zeros((256, 3), f32)], axis=1)
    colv = lambda v: v[:, None]
    wspec = lambda shp: pl.BlockSpec(shp, lambda i: (0, 0))
    eT = pl.pallas_call(
        _edge_body,
        grid=(_GRID,),
        in_specs=[pl.BlockSpec((8, _EB), lambda i: (0, i)),
                  wspec((256, 8)), wspec((256, 1)), wspec((256, 1)), wspec((256, 1)),
                  wspec((256, 256)), wspec((256, 1)), wspec((256, 1)), wspec((256, 1))],
        out_specs=pl.BlockSpec((256, _EB), lambda i: (0, i)),
        out_shape=jax.ShapeDtypeStruct((256, _NE), f32),
    )(X, w1e, colv(be1), colv(ge1), colv(oe1),
      We2.T, colv(be2), colv(ge2), colv(oe2))
    edge_features = eT.T[: _N * _K]

    g3 = gathered.reshape(_NPAD, _K, 128)
    mu = jnp.pad(input_means, (0, 50))[None, :]
    sg = jnp.pad(input_stds, (0, 50), constant_values=1.0)[None, :]
    w1n = jnp.concatenate([Wn1, jnp.zeros((50, 256), f32)], axis=0)
    rspec = lambda shp: pl.BlockSpec(shp, lambda i: (0, 0))
    n_out = pl.pallas_call(
        _node_body,
        grid=(_GRID,),
        in_specs=[pl.BlockSpec((_NB, _K), lambda i: (i, 0)),
                  pl.BlockSpec((_NB, _K, 128), lambda i: (i, 0, 0)),
                  rspec((1, 128)), rspec((1, 128)),
                  rspec((128, 256)), rspec((1, 256)), rspec((1, 256)), rspec((1, 256)),
                  rspec((256, 256)), rspec((1, 256)), rspec((1, 256)), rspec((1, 256))],
        out_specs=pl.BlockSpec((_NB, 256), lambda i: (i, 0)),
        out_shape=jax.ShapeDtypeStruct((_NPAD, 256), f32),
    )(d, g3, mu, sg, w1n, bn1[None, :], gn1[None, :], on1[None, :],
      Wn2, bn2[None, :], gn2[None, :], on2[None, :])
    n = n_out[:_N]

    senders = jnp.repeat(jnp.arange(_N), _K)
    receivers = jnp.arange(_N)
    return n, edge_features, senders, receivers


# speedup vs baseline: 256.4711x; 1.3801x over previous
import functools

import jax
import jax.numpy as jnp
from jax import lax
from jax.experimental import pallas as pl
from jax.experimental.pallas import tpu as pltpu
from jax.experimental.pallas import tpu_sc as plsc

_R_EARTH = 6371.0
_MASK_KM = 82.5
_N = 5978
_K = 32
_H = 361
_W = 720
_NPAD = 6016
_NB = 128
_GRID = _NPAD // _NB
_NROW = 9
_NCOL = 64
_WIN = 640
_NE = _NPAD * _K
_EB = _NB * _K


def _knn_body(slat_ref, cc_ref, rowb_ref, slon_ref, colf_ref, oi_ref, oa_ref):
    slon = slon_ref[...]
    colf = colf_ref[...]
    a_rows = [slat_ref[:, r:r + 1] + cc_ref[:, r:r + 1] * slon for r in range(_NROW)]
    a_rows.append(jnp.full((_NB, _NCOL), 1e9, jnp.float32))
    f_rows = [rowb_ref[:, r:r + 1] + colf for r in range(_NROW)]
    f_rows.append(400000.0 + colf)
    a = jnp.concatenate(a_rows, axis=1)
    fidx = jnp.concatenate(f_rows, axis=1)
    kiota = lax.broadcasted_iota(jnp.int32, (_NB, _K), 1)
    outi = jnp.zeros((_NB, _K), jnp.float32)
    outa = jnp.zeros((_NB, _K), jnp.float32)
    for k in range(_K):
        m = jnp.min(a, axis=1, keepdims=True)
        cand = jnp.where(a == m, fidx, 1e9)
        sel = jnp.min(cand, axis=1, keepdims=True)
        outi = jnp.where(kiota == k, sel, outi)
        outa = jnp.where(kiota == k, m, outa)
        a = jnp.where(fidx == sel, 1e30, a)
    oi_ref[...] = outi
    oa_ref[...] = outa


def _edge_body(x_ref, w1_ref, b1_ref, g1_ref, o1_ref,
               w2_ref, b2_ref, g2_ref, o2_ref, out_ref):
    x = x_ref[...]
    d = x[2:3, :]
    mask = (d <= _MASK_KM).astype(jnp.float32)
    c0 = x[0:1, :] - x[3:4, :]
    c1 = x[1:2, :] - x[4:5, :]
    c3 = jnp.cos(x[0:1, :] - x[5:6, :])
    c4 = jnp.sin(x[1:2, :] - x[6:7, :])
    zero = jnp.zeros_like(c0)
    c = jnp.concatenate([c0, c1, d, c3, c4, zero, zero, zero], axis=0)
    ct = jnp.transpose(c)
    maskt = jnp.transpose(mask)
    h = jnp.dot(ct, w1_ref[...], preferred_element_type=jnp.float32)
    h = jnp.maximum(h + b1_ref[...], 0.0)
    m1 = jnp.mean(h, axis=1, keepdims=True)
    v1 = jnp.mean((h - m1) ** 2, axis=1, keepdims=True)
    h = (h - m1) / jnp.sqrt(v1 + 1e-5) * g1_ref[...] + o1_ref[...]
    h = jnp.dot(h, w2_ref[...], preferred_element_type=jnp.float32)
    h = jnp.maximum(h + b2_ref[...], 0.0)
    m2 = jnp.mean(h, axis=1, keepdims=True)
    v2 = jnp.mean((h - m2) ** 2, axis=1, keepdims=True)
    h = (h - m2) / jnp.sqrt(v2 + 1e-5) * g2_ref[...] + o2_ref[...]
    out_ref[...] = h * maskt


def _node_body(d_ref, g_ref, mu_ref, sg_ref, w1_ref, b1_ref, g1_ref, o1_ref,
               w2_ref, b2_ref, g2_ref, o2_ref, out_ref):
    d = d_ref[...]
    mask = (d <= _MASK_KM).astype(jnp.float32)
    w = jnp.exp(-d / _MASK_KM) * mask
    sw = jnp.sum(w, axis=1, keepdims=True)
    w = w / (sw + 1e-7)
    swn = jnp.sum(w, axis=1, keepdims=True)
    agg = jnp.sum(g_ref[...] * w[:, :, None], axis=1)
    x = (agg - mu_ref[...] * swn) / (sg_ref[...] + 1e-7)
    h = jnp.dot(x, w1_ref[...], preferred_element_type=jnp.float32)
    h = jnp.maximum(h + b1_ref[...], 0.0)
    m1 = jnp.mean(h, axis=1, keepdims=True)
    v1 = jnp.mean((h - m1) ** 2, axis=1, keepdims=True)
    h = (h - m1) / jnp.sqrt(v1 + 1e-5) * g1_ref[...] + o1_ref[...]
    h = jnp.dot(h, w2_ref[...], preferred_element_type=jnp.float32)
    h = jnp.maximum(h + b2_ref[...], 0.0)
    m2 = jnp.mean(h, axis=1, keepdims=True)
    v2 = jnp.mean((h - m2) ** 2, axis=1, keepdims=True)
    h = (h - m2) / jnp.sqrt(v2 + 1e-5) * g2_ref[...] + o2_ref[...]
    out_ref[...] = h


def _sc_gather(table, idx):
    n_idx = idx.shape[0]
    n_workers = 32
    b_per_w = n_idx // n_workers
    chunk = 376
    n_chunks = b_per_w // chunk
    mesh = plsc.VectorSubcoreMesh(core_axis_name="c", subcore_axis_name="s")

    @functools.partial(
        pl.kernel, mesh=mesh,
        out_type=jax.ShapeDtypeStruct((n_idx, 128), jnp.float32),
        scratch_types=[
            pltpu.VMEM((chunk,), jnp.int32),
            pltpu.VMEM((chunk, 128), jnp.float32),
            pltpu.SemaphoreType.DMA,
        ],
    )
    def k(table_hbm, idx_hbm, out_hbm, idx_v, rows_v, sem):
        wid = lax.axis_index("s") * 2 + lax.axis_index("c")
        base = wid * b_per_w

        @pl.loop(0, n_chunks)
        def _(i):
            off = base + i * chunk
            pltpu.sync_copy(idx_hbm.at[pl.ds(off, chunk)], idx_v)
            pltpu.async_copy(table_hbm.at[idx_v], rows_v, sem).wait()
            pltpu.sync_copy(rows_v, out_hbm.at[pl.ds(off, chunk)])

    return k(table, idx)


def kernel(combined_data, ico_positions, input_means, input_stds,
           We1, be1, ge1, oe1, We2, be2, ge2, oe2,
           Wn1, bn1, gn1, on1, Wn2, bn2, gn2, on2):
    f32 = jnp.float32
    lat_grid = jnp.radians(jnp.linspace(-90.0, 90.0, _H, dtype=f32))
    lon_grid = jnp.radians(jnp.linspace(0.0, 360.0, _W, dtype=f32))
    ico_lat = jnp.radians(ico_positions[:, 0])
    ico_lon = jnp.radians(ico_positions[:, 1])
    lat_deg = ico_positions[:, 0]
    lon_deg = ico_positions[:, 1]
    crow = jnp.round((lat_deg + 90.0) * 2.0).astype(jnp.int32)
    start = jnp.clip(crow - 4, 0, _H - _NROW)
    ccol = jnp.clip(jnp.round(lon_deg * (719.0 / 360.0)).astype(jnp.int32), 0, _W - 1)
    colmap = jnp.mod(ccol[:, None] - 31 + jnp.arange(63, dtype=jnp.int32)[None, :], 719)
    colmap = jnp.concatenate(
        [colmap, jnp.full((_N, 1), _W - 1, jnp.int32)], axis=1)
    rows = start[:, None] + jnp.arange(_NROW, dtype=jnp.int32)[None, :]
    llatw = jnp.take(lat_grid, rows)
    slat2w = jnp.sin((llatw - ico_lat[:, None]) / 2) ** 2
    ccw = jnp.cos(ico_lat)[:, None] * jnp.cos(llatw)
    llonw = jnp.take(lon_grid, colmap)
    slon2w = jnp.sin((llonw - ico_lon[:, None]) / 2) ** 2

    def pad16(x, fill):
        return jnp.pad(x, ((0, 0), (0, 16 - _NROW)), constant_values=fill)

    slat_t = pad16(slat2w, 1e9)
    cc_t = pad16(ccw, 0.0)
    rowb_t = pad16((rows * _W).astype(f32), 500000.0)
    colf = colmap.astype(f32)

    def padn(x):
        return jnp.pad(x, ((0, _NPAD - _N),) + ((0, 0),) * (x.ndim - 1), mode="edge")

    slat_t, cc_t, rowb_t, slon_t, colf_t = map(padn, (slat_t, cc_t, rowb_t, slon2w, colf))

    bspec16 = pl.BlockSpec((_NB, 16), lambda i: (i, 0))
    bspec64 = pl.BlockSpec((_NB, _NCOL), lambda i: (i, 0))
    bspec_out = pl.BlockSpec((_NB, _K), lambda i: (i, 0))
    sel_f, sel_a = pl.pallas_call(
        _knn_body,
        grid=(_GRID,),
        in_specs=[bspec16, bspec16, bspec16, bspec64, bspec64],
        out_specs=[bspec_out, bspec_out],
        out_shape=[jax.ShapeDtypeStruct((_NPAD, _K), f32)] * 2,
    )(slat_t, cc_t, rowb_t, slon_t, colf_t)

    fi = sel_f.astype(jnp.int32)
    row = fi // _W
    col = fi - row * _W
    llat = jnp.take(lat_grid, row)
    llon = jnp.take(lon_grid, col)
    central = 2.0 * jnp.arcsin(jnp.sqrt(jnp.clip(sel_a, 0.0, 1.0)))
    d = central * _R_EARTH

    table = jnp.pad(combined_data.reshape(78, -1), ((0, 50), (0, 0))).T
    gathered = _sc_gather(table, fi.reshape(-1))

    deg_la = jnp.degrees(ico_lat)
    deg_lo = jnp.degrees(ico_lon)

    def pedge(node_arr):
        return jnp.repeat(jnp.pad(node_arr, (0, _NPAD - _N), mode="edge"), _K)

    X = jnp.stack([llat.reshape(-1), llon.reshape(-1), d.reshape(-1),
                   pedge(deg_la), pedge(deg_lo), pedge(ico_lat), pedge(ico_lon),
                   jnp.zeros((_NE,), f32)], axis=0)
    w1e = jnp.concatenate([We1, jnp.zeros((3, 256), f32)], axis=0)
    rowv = lambda v: v[None, :]
    wspec = lambda shp: pl.BlockSpec(shp, lambda i: (0, 0))
    ef = pl.pallas_call(
        _edge_body,
        grid=(_GRID,),
        in_specs=[pl.BlockSpec((8, _EB), lambda i: (0, i)),
                  wspec((8, 256)), wspec((1, 256)), wspec((1, 256)), wspec((1, 256)),
                  wspec((256, 256)), wspec((1, 256)), wspec((1, 256)), wspec((1, 256))],
        out_specs=pl.BlockSpec((_EB, 256), lambda i: (i, 0)),
        out_shape=jax.ShapeDtypeStruct((_NE, 256), f32),
    )(X, w1e, rowv(be1), rowv(ge1), rowv(oe1),
      We2, rowv(be2), rowv(ge2), rowv(oe2))
    edge_features = ef[: _N * _K]

    g3 = gathered.reshape(_NPAD, _K, 128)
    mu = jnp.pad(input_means, (0, 50))[None, :]
    sg = jnp.pad(input_stds, (0, 50), constant_values=1.0)[None, :]
    w1n = jnp.concatenate([Wn1, jnp.zeros((50, 256), f32)], axis=0)
    rspec = lambda shp: pl.BlockSpec(shp, lambda i: (0, 0))
    n_out = pl.pallas_call(
        _node_body,
        grid=(_GRID,),
        in_specs=[pl.BlockSpec((_NB, _K), lambda i: (i, 0)),
                  pl.BlockSpec((_NB, _K, 128), lambda i: (i, 0, 0)),
                  rspec((1, 128)), rspec((1, 128)),
                  rspec((128, 256)), rspec((1, 256)), rspec((1, 256)), rspec((1, 256)),
                  rspec((256, 256)), rspec((1, 256)), rspec((1, 256)), rspec((1, 256))],
        out_specs=pl.BlockSpec((_NB, 256), lambda i: (i, 0)),
        out_shape=jax.ShapeDtypeStruct((_NPAD, 256), f32),
    )(d, g3, mu, sg, w1n, bn1[None, :], gn1[None, :], on1[None, :],
      Wn2, bn2[None, :], gn2[None, :], on2[None, :])
    n = n_out[:_N]

    senders = jnp.repeat(jnp.arange(_N), _K)
    receivers = jnp.arange(_N)
    return n, edge_features, senders, receivers


# speedup vs baseline: 556.2516x; 2.1689x over previous
import functools

import jax
import jax.numpy as jnp
from jax import lax
from jax.experimental import pallas as pl
from jax.experimental.pallas import tpu as pltpu
from jax.experimental.pallas import tpu_sc as plsc

_R_EARTH = 6371.0
_MASK_KM = 82.5
_N = 5978
_K = 32
_H = 361
_W = 720
_NPAD = 6016
_NB = 128
_GRID = _NPAD // _NB
_NROW = 9
_NCOL = 64
_WIN = 640
_NE = _NPAD * _K
_EB = _NB * _K


def _knn_body(slat_ref, cc_ref, rowb_ref, slon_ref, colf_ref, oi_ref, oa_ref):
    slon = slon_ref[...]
    colf = colf_ref[...]
    a_rows = [slat_ref[:, r:r + 1] + cc_ref[:, r:r + 1] * slon for r in range(_NROW)]
    a_rows.append(jnp.full((_NB, _NCOL), 1e9, jnp.float32))
    f_rows = [rowb_ref[:, r:r + 1] + colf for r in range(_NROW)]
    f_rows.append(400000.0 + colf)
    a = jnp.concatenate(a_rows, axis=1)
    fidx = jnp.concatenate(f_rows, axis=1)
    kiota = lax.broadcasted_iota(jnp.int32, (_NB, _K), 1)
    outi = jnp.zeros((_NB, _K), jnp.float32)
    outa = jnp.zeros((_NB, _K), jnp.float32)
    for k in range(_K):
        m = jnp.min(a, axis=1, keepdims=True)
        cand = jnp.where(a == m, fidx, 1e9)
        sel = jnp.min(cand, axis=1, keepdims=True)
        outi = jnp.where(kiota == k, sel, outi)
        outa = jnp.where(kiota == k, m, outa)
        a = jnp.where(fidx == sel, 1e30, a)
    oi_ref[...] = outi
    oa_ref[...] = outa


_DEG2RAD = 0.017453292519943295
_LONSTEP = 6.283185307179586 / 719.0


def _edge_body(x_ref, w1_ref, b1_ref, g1_ref, o1_ref,
               w2_ref, b2_ref, g2_ref, o2_ref, out_ref):
    x = x_ref[...]
    fi = x[0:1, :]
    d = x[1:2, :]
    rowf = jnp.floor((fi + 0.5) * (1.0 / _W))
    colf = fi - _W * rowf
    llat = (rowf * 0.5 - 90.0) * _DEG2RAD
    llon = colf * _LONSTEP
    la_deg = x[2:3, :]
    lo_deg = x[3:4, :]
    mask = (d <= _MASK_KM).astype(jnp.float32)
    c0 = llat - la_deg
    c1 = llon - lo_deg
    c3 = jnp.cos(llat - la_deg * _DEG2RAD)
    c4 = jnp.sin(llon - lo_deg * _DEG2RAD)
    zero = jnp.zeros_like(c0)
    c = jnp.concatenate([c0, c1, d, c3, c4, zero, zero, zero], axis=0)
    ct = jnp.transpose(c)
    maskt = jnp.transpose(mask)
    h = jnp.dot(ct, w1_ref[...], preferred_element_type=jnp.float32)
    h = jnp.maximum(h + b1_ref[...], 0.0)
    m1 = jnp.mean(h, axis=1, keepdims=True)
    v1 = jnp.mean((h - m1) ** 2, axis=1, keepdims=True)
    h = (h - m1) / jnp.sqrt(v1 + 1e-5) * g1_ref[...] + o1_ref[...]
    h = jnp.dot(h, w2_ref[...], preferred_element_type=jnp.float32)
    h = jnp.maximum(h + b2_ref[...], 0.0)
    m2 = jnp.mean(h, axis=1, keepdims=True)
    v2 = jnp.mean((h - m2) ** 2, axis=1, keepdims=True)
    h = (h - m2) / jnp.sqrt(v2 + 1e-5) * g2_ref[...] + o2_ref[...]
    out_ref[...] = h * maskt


def _node_body(a_ref, g_ref, mu_ref, sg_ref, w1_ref, b1_ref, g1_ref, o1_ref,
               w2_ref, b2_ref, g2_ref, o2_ref, out_ref):
    d = a_ref[...]
    mask = (d <= _MASK_KM).astype(jnp.float32)
    w = jnp.exp(-d / _MASK_KM) * mask
    sw = jnp.sum(w, axis=1, keepdims=True)
    w = w / (sw + 1e-7)
    swn = jnp.sum(w, axis=1, keepdims=True)
    agg = jnp.sum(g_ref[...] * w[:, :, None], axis=1)
    x = (agg - mu_ref[...] * swn) / (sg_ref[...] + 1e-7)
    h = jnp.dot(x, w1_ref[...], preferred_element_type=jnp.float32)
    h = jnp.maximum(h + b1_ref[...], 0.0)
    m1 = jnp.mean(h, axis=1, keepdims=True)
    v1 = jnp.mean((h - m1) ** 2, axis=1, keepdims=True)
    h = (h - m1) / jnp.sqrt(v1 + 1e-5) * g1_ref[...] + o1_ref[...]
    h = jnp.dot(h, w2_ref[...], preferred_element_type=jnp.float32)
    h = jnp.maximum(h + b2_ref[...], 0.0)
    m2 = jnp.mean(h, axis=1, keepdims=True)
    v2 = jnp.mean((h - m2) ** 2, axis=1, keepdims=True)
    h = (h - m2) / jnp.sqrt(v2 + 1e-5) * g2_ref[...] + o2_ref[...]
    out_ref[...] = h


def _sc_gather(table, idx):
    n_idx = idx.shape[0]
    n_workers = 32
    b_per_w = n_idx // n_workers
    chunk = 376
    n_chunks = b_per_w // chunk
    mesh = plsc.VectorSubcoreMesh(core_axis_name="c", subcore_axis_name="s")

    @functools.partial(
        pl.kernel, mesh=mesh,
        out_type=jax.ShapeDtypeStruct((n_idx, 128), jnp.float32),
        scratch_types=[
            pltpu.VMEM((chunk,), jnp.int32),
            pltpu.VMEM((chunk, 128), jnp.float32),
            pltpu.SemaphoreType.DMA,
        ],
    )
    def k(table_hbm, idx_hbm, out_hbm, idx_v, rows_v, sem):
        wid = lax.axis_index("s") * 2 + lax.axis_index("c")
        base = wid * b_per_w

        @pl.loop(0, n_chunks)
        def _(i):
            off = base + i * chunk
            pltpu.sync_copy(idx_hbm.at[pl.ds(off, chunk)], idx_v)
            pltpu.async_copy(table_hbm.at[idx_v], rows_v, sem).wait()
            pltpu.sync_copy(rows_v, out_hbm.at[pl.ds(off, chunk)])

    return k(table, idx)


def kernel(combined_data, ico_positions, input_means, input_stds,
           We1, be1, ge1, oe1, We2, be2, ge2, oe2,
           Wn1, bn1, gn1, on1, Wn2, bn2, gn2, on2):
    f32 = jnp.float32
    lat_grid = jnp.radians(jnp.linspace(-90.0, 90.0, _H, dtype=f32))
    lon_grid = jnp.radians(jnp.linspace(0.0, 360.0, _W, dtype=f32))
    ico_lat = jnp.radians(ico_positions[:, 0])
    ico_lon = jnp.radians(ico_positions[:, 1])
    lat_deg = ico_positions[:, 0]
    lon_deg = ico_positions[:, 1]
    crow = jnp.round((lat_deg + 90.0) * 2.0).astype(jnp.int32)
    start = jnp.clip(crow - 4, 0, _H - _NROW)
    ccol = jnp.clip(jnp.round(lon_deg * (719.0 / 360.0)).astype(jnp.int32), 0, _W - 1)
    colmap = jnp.mod(ccol[:, None] - 31 + jnp.arange(63, dtype=jnp.int32)[None, :], 719)
    colmap = jnp.concatenate(
        [colmap, jnp.full((_N, 1), _W - 1, jnp.int32)], axis=1)
    rows = start[:, None] + jnp.arange(_NROW, dtype=jnp.int32)[None, :]
    llatw = jnp.take(lat_grid, rows)
    slat2w = jnp.sin((llatw - ico_lat[:, None]) / 2) ** 2
    ccw = jnp.cos(ico_lat)[:, None] * jnp.cos(llatw)
    llonw = jnp.take(lon_grid, colmap)
    slon2w = jnp.sin((llonw - ico_lon[:, None]) / 2) ** 2

    def pad16(x, fill):
        return jnp.pad(x, ((0, 0), (0, 16 - _NROW)), constant_values=fill)

    slat_t = pad16(slat2w, 1e9)
    cc_t = pad16(ccw, 0.0)
    rowb_t = pad16((rows * _W).astype(f32), 500000.0)
    colf = colmap.astype(f32)

    def padn(x):
        return jnp.pad(x, ((0, _NPAD - _N),) + ((0, 0),) * (x.ndim - 1), mode="edge")

    slat_t, cc_t, rowb_t, slon_t, colf_t = map(padn, (slat_t, cc_t, rowb_t, slon2w, colf))

    bspec16 = pl.BlockSpec((_NB, 16), lambda i: (i, 0))
    bspec64 = pl.BlockSpec((_NB, _NCOL), lambda i: (i, 0))
    bspec_out = pl.BlockSpec((_NB, _K), lambda i: (i, 0))
    sel_f, sel_a = pl.pallas_call(
        _knn_body,
        grid=(_GRID,),
        in_specs=[bspec16, bspec16, bspec16, bspec64, bspec64],
        out_specs=[bspec_out, bspec_out],
        out_shape=[jax.ShapeDtypeStruct((_NPAD, _K), f32)] * 2,
    )(slat_t, cc_t, rowb_t, slon_t, colf_t)

    fi = sel_f.astype(jnp.int32)
    d = 2.0 * jnp.arcsin(jnp.sqrt(jnp.clip(sel_a, 0.0, 1.0))) * _R_EARTH

    table = jnp.pad(combined_data.reshape(78, -1), ((0, 50), (0, 0))).T
    gathered = _sc_gather(table, fi.reshape(-1))

    def pedge(node_arr):
        return jnp.repeat(jnp.pad(node_arr, (0, _NPAD - _N), mode="edge"), _K)

    X = jnp.stack([sel_f.reshape(-1), d.reshape(-1),
                   pedge(lat_deg), pedge(lon_deg)], axis=0)
    w1e = jnp.concatenate([We1, jnp.zeros((3, 256), f32)], axis=0)
    rowv = lambda v: v[None, :]
    wspec = lambda shp: pl.BlockSpec(shp, lambda i: (0, 0))
    ef = pl.pallas_call(
        _edge_body,
        grid=(_GRID,),
        in_specs=[pl.BlockSpec((4, _EB), lambda i: (0, i)),
                  wspec((8, 256)), wspec((1, 256)), wspec((1, 256)), wspec((1, 256)),
                  wspec((256, 256)), wspec((1, 256)), wspec((1, 256)), wspec((1, 256))],
        out_specs=pl.BlockSpec((_EB, 256), lambda i: (i, 0)),
        out_shape=jax.ShapeDtypeStruct((_N * _K, 256), f32),
    )(X, w1e, rowv(be1), rowv(ge1), rowv(oe1),
      We2, rowv(be2), rowv(ge2), rowv(oe2))
    edge_features = ef

    g3 = gathered.reshape(_NPAD, _K, 128)
    mu = jnp.pad(input_means, (0, 50))[None, :]
    sg = jnp.pad(input_stds, (0, 50), constant_values=1.0)[None, :]
    w1n = jnp.concatenate([Wn1, jnp.zeros((50, 256), f32)], axis=0)
    rspec = lambda shp: pl.BlockSpec(shp, lambda i: (0, 0))
    n_out = pl.pallas_call(
        _node_body,
        grid=(_GRID,),
        in_specs=[pl.BlockSpec((_NB, _K), lambda i: (i, 0)),
                  pl.BlockSpec((_NB, _K, 128), lambda i: (i, 0, 0)),
                  rspec((1, 128)), rspec((1, 128)),
                  rspec((128, 256)), rspec((1, 256)), rspec((1, 256)), rspec((1, 256)),
                  rspec((256, 256)), rspec((1, 256)), rspec((1, 256)), rspec((1, 256))],
        out_specs=pl.BlockSpec((_NB, 256), lambda i: (i, 0)),
        out_shape=jax.ShapeDtypeStruct((_N, 256), f32),
    )(d, g3, mu, sg, w1n, bn1[None, :], gn1[None, :], on1[None, :],
      Wn2, bn2[None, :], gn2[None, :], on2[None, :])
    n = n_out

    senders = jnp.repeat(jnp.arange(_N), _K)
    receivers = jnp.arange(_N)
    return n, edge_features, senders, receivers
